# Initial kernel scaffold; baseline (speedup 1.0000x reference)
import jax
import jax.numpy as jnp
from jax.experimental import pallas as pl


def kernel(node_feat, node_opcode, edge_index, config_feat, n_configs, batch, op_emb, set_emb, W0, att_src0, att_dst0, bias0, W1, att_src1, att_dst1, bias1, W2, att_src2, att_dst2, bias2, pW1, pb1, pW2, pb2):
    raise NotImplementedError("write your pallas kernel here")


# speedup vs baseline: 4.0607x; 4.0607x over previous
import jax
import jax.numpy as jnp
from jax.experimental import pallas as pl
from jax.experimental.pallas import tpu as pltpu

N_NODES = 10000
N_OPS = 120
N_HEADS = 4
H_DIM = 64
HID = N_HEADS * H_DIM
N_GRAPHS = 16
ECHUNK = 512
NT = 1000


def _head_expand(att):
    a = att.reshape(-1)
    grp = jnp.arange(HID) // H_DIM
    blk = (grp[:, None] == grp[None, :]).astype(jnp.float32)
    return a[:, None] * blk


def _k1a_body(nf_ref, opc_ref, wn_ref, ow_ref, sw_ref, asx_ref, adx_ref,
              h_ref, asn_ref, adn_ref):
    nf = nf_ref[...]
    opc = opc_ref[...]
    oh_op = (jax.lax.broadcasted_iota(jnp.int32, (NT, N_OPS), 1)
             == opc).astype(jnp.float32)
    sidx = nf[:, 139:140].astype(jnp.int32)
    oh_set = (jax.lax.broadcasted_iota(jnp.int32, (NT, 8), 1)
              == sidx).astype(jnp.float32)
    h = jnp.dot(nf, wn_ref[...], preferred_element_type=jnp.float32)
    h = h + jnp.dot(oh_op, ow_ref[...], preferred_element_type=jnp.float32)
    h = h + jnp.dot(oh_set, sw_ref[...], preferred_element_type=jnp.float32)
    h_ref[...] = h
    asn = jnp.dot(h, asx_ref[...], preferred_element_type=jnp.float32)
    adn = jnp.dot(h, adx_ref[...], preferred_element_type=jnp.float32)
    asn_ref[...] = asn
    adn_ref[...] = adn


def _k1b_body(agg_ref, bias_ref, w_ref, asx_ref, adx_ref,
              h_ref, asn_ref, adn_ref):
    z = agg_ref[...] + bias_ref[...]
    x = jnp.where(z > 0, z, (jnp.exp(z) - 1.0))
    h = jnp.dot(x, w_ref[...], preferred_element_type=jnp.float32)
    h_ref[...] = h
    asn = jnp.dot(h, asx_ref[...], preferred_element_type=jnp.float32)
    adn = jnp.dot(h, adx_ref[...], preferred_element_type=jnp.float32)
    asn_ref[...] = asn
    adn_ref[...] = adn


def _k2_body(src_ref, dst_ref, asn_ref, adn_ref, den_ref, bscr):
    i = pl.program_id(0)

    @pl.when(i == 0)
    def _():
        den_ref[...] = jnp.zeros_like(den_ref)
        bscr[0, 0] = jnp.maximum(
            jnp.max(asn_ref[...]) + jnp.max(adn_ref[...]), 0.0)

    b = bscr[0, 0]

    def body(j, carry):
        s = src_ref[0, 0, j]
        d = dst_ref[0, 0, j]
        z = asn_ref[pl.ds(s, 1), :] + adn_ref[pl.ds(d, 1), :]
        e = jnp.where(z >= 0, z, 0.2 * z)
        ex = jnp.exp(e - b)
        den_ref[pl.ds(d, 1), :] += ex
        return carry

    jax.lax.fori_loop(0, ECHUNK, body, 0)


def _k3_body(src_ref, dst_ref, asn_ref, adn_ref, h_ref, den_ref,
             out_ref, bscr):
    i = pl.program_id(0)

    @pl.when(i == 0)
    def _():
        out_ref[...] = jnp.zeros_like(out_ref)
        bscr[0, 0] = jnp.maximum(
            jnp.max(asn_ref[...]) + jnp.max(adn_ref[...]), 0.0)

    b = bscr[0, 0]

    def body(j, carry):
        s = src_ref[0, 0, j]
        d = dst_ref[0, 0, j]
        z = asn_ref[pl.ds(s, 1), :] + adn_ref[pl.ds(d, 1), :]
        e = jnp.where(z >= 0, z, 0.2 * z)
        ex = jnp.exp(e - b)
        alpha = ex / (den_ref[pl.ds(d, 1), :] + 1e-16)
        out_ref[pl.ds(d, 1), :] += h_ref[pl.ds(s, 1), :] * alpha
        return carry

    jax.lax.fori_loop(0, ECHUNK, body, 0)


def _k4_body(agg_ref, bias_ref, batch_ref, rep_ref, cf_ref, w1a_ref,
             w1b_ref, b1_ref, w2_ref, b2_ref, out_ref):
    z = agg_ref[...] + bias_ref[...]
    x = jnp.where(z > 0, z, (jnp.exp(z) - 1.0))
    oh_b = (jax.lax.broadcasted_iota(jnp.int32, (N_GRAPHS, N_NODES), 0)
            == batch_ref[...]).astype(jnp.float32)
    pooled = jnp.dot(oh_b, x, preferred_element_type=jnp.float32)
    xg = jnp.dot(rep_ref[...], pooled, preferred_element_type=jnp.float32)
    hdd = (jnp.dot(xg, w1a_ref[...], preferred_element_type=jnp.float32)
           + jnp.dot(cf_ref[...], w1b_ref[...],
                     preferred_element_type=jnp.float32)
           + b1_ref[...])
    hdd = jnp.maximum(hdd, 0.0)
    out_ref[...] = (jnp.dot(hdd, w2_ref[...],
                            preferred_element_type=jnp.float32)
                    + b2_ref[...])


def _node_call(body, tiled_shapes, *args):
    in_specs = []
    for a, shp in zip(args, tiled_shapes):
        if shp is None:
            in_specs.append(pl.BlockSpec(a.shape, lambda i: (0, 0)))
        else:
            in_specs.append(pl.BlockSpec(shp, lambda i: (i, 0)))
    outs = [jax.ShapeDtypeStruct((N_NODES, HID), jnp.float32)] * 3
    out_specs = [pl.BlockSpec((NT, HID), lambda i: (i, 0))] * 3
    return pl.pallas_call(
        body,
        grid=(N_NODES // NT,),
        in_specs=in_specs,
        out_shape=outs,
        out_specs=out_specs,
    )(*args)


def _edge_layer(srcc, dstc, asn, adn, h):
    nchunk = srcc.shape[0]
    idx_spec = pl.BlockSpec((1, 1, ECHUNK), lambda i: (i, 0, 0),
                            memory_space=pltpu.SMEM)
    node_spec = pl.BlockSpec((N_NODES, HID), lambda i: (0, 0))
    scr = [pltpu.SMEM((1, 1), jnp.float32)]
    den = pl.pallas_call(
        _k2_body,
        grid=(nchunk,),
        in_specs=[idx_spec, idx_spec, node_spec, node_spec],
        out_specs=node_spec,
        out_shape=jax.ShapeDtypeStruct((N_NODES, HID), jnp.float32),
        scratch_shapes=scr,
    )(srcc, dstc, asn, adn)
    out = pl.pallas_call(
        _k3_body,
        grid=(nchunk,),
        in_specs=[idx_spec, idx_spec, node_spec, node_spec,
                  node_spec, node_spec],
        out_specs=node_spec,
        out_shape=jax.ShapeDtypeStruct((N_NODES, HID), jnp.float32),
        scratch_shapes=scr,
    )(srcc, dstc, asn, adn, h, den)
    return out


def kernel(node_feat, node_opcode, edge_index, config_feat, n_configs,
           batch, op_emb, set_emb,
           W0, att_src0, att_dst0, bias0,
           W1, att_src1, att_dst1, bias1,
           W2, att_src2, att_dst2, bias2,
           pW1, pb1, pW2, pb2):
    f32 = jnp.float32
    src0 = edge_index[:, 0].astype(jnp.int32)
    dst0 = edge_index[:, 1].astype(jnp.int32)
    src = jnp.concatenate([src0, dst0])
    dst = jnp.concatenate([dst0, src0])
    e2 = src.shape[0]
    nchunk = e2 // ECHUNK
    srcc = src.reshape(nchunk, 1, ECHUNK)
    dstc = dst.reshape(nchunk, 1, ECHUNK)

    W0 = W0.astype(f32)
    Wn = jnp.concatenate([W0[:139], jnp.zeros((1, HID), f32)], axis=0)
    OW = op_emb.astype(f32) @ W0[139:147]
    SW = set_emb.astype(f32) @ W0[147:151]
    exps = [(_head_expand(att_src0), _head_expand(att_dst0)),
            (_head_expand(att_src1), _head_expand(att_dst1)),
            (_head_expand(att_src2), _head_expand(att_dst2))]

    opc = node_opcode.astype(jnp.int32).reshape(N_NODES, 1)
    nf = node_feat.astype(f32)

    h, asn, adn = _node_call(
        _k1a_body, [(NT, 140), (NT, 1), None, None, None, None, None],
        nf, opc, Wn, OW, SW, exps[0][0], exps[0][1])
    agg = _edge_layer(srcc, dstc, asn, adn, h)

    for l, (Wl, biasl) in enumerate([(W1, bias0), (W2, bias1)]):
        h, asn, adn = _node_call(
            _k1b_body, [(NT, HID), None, None, None, None], agg,
            biasl.reshape(1, HID).astype(f32), Wl.astype(f32),
            exps[l + 1][0], exps[l + 1][1])
        agg = _edge_layer(srcc, dstc, asn, adn, h)

    total = config_feat.shape[0]
    gid = jnp.repeat(jnp.arange(N_GRAPHS), n_configs,
                     total_repeat_length=total)
    rep = (gid[:, None] == jnp.arange(N_GRAPHS)[None, :]).astype(f32)
    batch2 = batch.astype(jnp.int32).reshape(1, N_NODES)
    out = pl.pallas_call(
        _k4_body,
        out_shape=jax.ShapeDtypeStruct((total, 1), f32),
    )(agg, bias2.reshape(1, HID).astype(f32), batch2, rep,
      config_feat.astype(f32), pW1[:HID].astype(f32),
      pW1[HID:].astype(f32), pb1.reshape(1, H_DIM).astype(f32),
      pW2.astype(f32), pb2.reshape(1, 1).astype(f32))
    return out[:, 0]
